# baseline (speedup 1.0000x reference)
import functools

import jax
import jax.numpy as jnp
from jax import lax
from jax.experimental import pallas as pl
from jax.experimental.pallas import tpu as pltpu
from jax.experimental.pallas import tpu_sc as plsc

B, V, E = 1024, 100000, 16

CHV = 32
V_TC = 0
NCHT = V // CHV
CH0 = V_TC // CHV
N_SC_CH = NCHT - CH0

VC = 2048
NV_TC = (V_TC + VC - 1) // VC

NC, NS = 2, 16
NT = NC * NS
NCH_LO = N_SC_CH // NT
NCH_REM = N_SC_CH - NCH_LO * NT
G = 2
NGB = (B // 16) // G
ER = E * CHV // 128

_sc_mesh = plsc.VectorSubcoreMesh(core_axis_name="c", subcore_axis_name="s")


def _bcast(vec, ilane):
    return lax.gather(
        vec, ilane[:, None],
        lax.GatherDimensionNumbers(offset_dims=(), collapsed_slice_dims=(0,),
                                   start_index_map=(0,)),
        slice_sizes=(1,),
        mode=lax.GatherScatterMode.PROMISE_IN_BOUNDS)


@functools.partial(
    pl.kernel,
    out_type=jax.ShapeDtypeStruct((NT, E, B), jnp.float32),
    mesh=_sc_mesh,
    scratch_types=[
        pltpu.VMEM((E, B), jnp.float32),
        pltpu.VMEM((2, CHV, B), jnp.float32),
        pltpu.VMEM((2, ER, 128), jnp.float32),
        pltpu.SemaphoreType.DMA,
        pltpu.SemaphoreType.DMA,
        pltpu.SemaphoreType.DMA,
        pltpu.SemaphoreType.DMA,
    ],
)
def _sc_embed(emb_c_hbm, x_t_hbm, out_hbm, acc_ref, x_ref, e_ref,
              xsem0, xsem1, esem0, esem1):
    wid = lax.axis_index("s") * NC + lax.axis_index("c")
    base_ch = CH0 + wid * NCH_LO + jnp.minimum(wid, NCH_REM)
    n_ch = NCH_LO + jnp.where(wid < NCH_REM, 1, 0)

    def _start(c, buf, xsem, esem):
        j = base_ch + c
        pltpu.async_copy(x_t_hbm.at[pl.ds(j * CHV, CHV), :],
                         x_ref.at[buf], xsem)
        pltpu.async_copy(emb_c_hbm.at[j], e_ref.at[buf], esem)

    def _start_p(c, buf):
        @pl.when(buf == 0)
        def _():
            _start(c, 0, xsem0, esem0)

        @pl.when(buf == 1)
        def _():
            _start(c, 1, xsem1, esem1)

    def _wait_p(buf):
        @pl.when(buf == 0)
        def _():
            pltpu.make_async_copy(x_t_hbm.at[pl.ds(0, CHV), :],
                                  x_ref.at[0], xsem0).wait()
            pltpu.make_async_copy(emb_c_hbm.at[0], e_ref.at[0], esem0).wait()

        @pl.when(buf == 1)
        def _():
            pltpu.make_async_copy(x_t_hbm.at[pl.ds(0, CHV), :],
                                  x_ref.at[1], xsem1).wait()
            pltpu.make_async_copy(emb_c_hbm.at[0], e_ref.at[1], esem1).wait()

    zero = jnp.zeros((16,), jnp.float32)

    def _zero(g, carry):
        for e in range(E):
            acc_ref[e, pl.ds(g * 16, 16)] = zero
        return carry

    _start_p(0, 0)
    lax.fori_loop(0, B // 16, _zero, 0)

    def _chunk(c, carry):
        buf = lax.rem(c, 2)

        @pl.when(c + 1 < n_ch)
        def _():
            _start_p(c + 1, 1 - buf)

        _wait_p(buf)

        def _gblk(gb, carry2):
            b0 = gb * (G * 16)
            accs = tuple(acc_ref[e, pl.ds(b0 + g * 16, 16)]
                         for e in range(E) for g in range(G))

            for vb in range(CHV // 16):

                def _v(v, accs, vb=vb):
                    ilane = jnp.full((16,), v - vb * 16, jnp.int32)
                    xvs = [x_ref[buf, v, pl.ds(b0 + g * 16, 16)]
                           for g in range(G)]
                    new = []
                    i = 0
                    for e in range(E):
                        ev = e_ref[buf, (e * CHV + vb * 16) // 128,
                                   pl.ds((e * CHV + vb * 16) % 128, 16)]
                        s = _bcast(ev, ilane)
                        for g in range(G):
                            new.append(accs[i] + xvs[g] * s)
                            i += 1
                    return tuple(new)

                accs = lax.fori_loop(vb * 16, vb * 16 + 16, _v, accs, unroll=4)

            i = 0
            for e in range(E):
                for g in range(G):
                    acc_ref[e, pl.ds(b0 + g * 16, 16)] = accs[i]
                    i += 1
            return carry2

        lax.fori_loop(0, NGB, _gblk, 0)
        return carry

    lax.fori_loop(0, n_ch, _chunk, 0)

    pltpu.sync_copy(acc_ref, out_hbm.at[wid])


def _tc_body(e_ref, x_ref, o_ref):
    k = pl.program_id(0)

    @pl.when(k == 0)
    def _():
        o_ref[...] = jnp.zeros_like(o_ref)

    col = jax.lax.broadcasted_iota(jnp.int32, (E, VC), 1)
    em = jnp.where(k * VC + col < V_TC, e_ref[...], 0.0)
    o_ref[...] += jnp.dot(em, x_ref[...], preferred_element_type=jnp.float32)


def _tc_matmul(emb_t, x_t):
    return pl.pallas_call(
        _tc_body,
        grid=(NV_TC,),
        in_specs=[
            pl.BlockSpec((E, VC), lambda k: (0, k)),
            pl.BlockSpec((VC, B), lambda k: (k, 0)),
        ],
        out_specs=pl.BlockSpec((E, B), lambda k: (0, 0)),
        out_shape=jax.ShapeDtypeStruct((E, B), jnp.float32),
        compiler_params=pltpu.CompilerParams(
            dimension_semantics=("arbitrary",),
        ),
    )(emb_t, x_t)


@jax.jit
def kernel(x, embedding):
    emb_c = (embedding.T.reshape(E, NCHT, CHV).transpose(1, 0, 2)
             .reshape(NCHT, ER, 128))
    out_t = _sc_embed(emb_c, x.T).sum(axis=0)
    if V_TC:
        out_t = out_t + _tc_matmul(embedding.T, x.T)
    return out_t.T


# speedup vs baseline: 25.4375x; 25.4375x over previous
import functools

import jax
import jax.numpy as jnp
from jax import lax
from jax.experimental import pallas as pl
from jax.experimental.pallas import tpu as pltpu
from jax.experimental.pallas import tpu_sc as plsc

B, V, E = 1024, 100000, 16

CHV = 32
V_TC = 95904
NCHT = V // CHV
CH0 = V_TC // CHV
N_SC_CH = NCHT - CH0

VC = 2048
NV_TC = (V_TC + VC - 1) // VC

NC, NS = 2, 16
NT = NC * NS
NCH_LO = N_SC_CH // NT
NCH_REM = N_SC_CH - NCH_LO * NT
G = 2
NGB = (B // 16) // G
ER = E * CHV // 128

_sc_mesh = plsc.VectorSubcoreMesh(core_axis_name="c", subcore_axis_name="s")


def _bcast(vec, ilane):
    return lax.gather(
        vec, ilane[:, None],
        lax.GatherDimensionNumbers(offset_dims=(), collapsed_slice_dims=(0,),
                                   start_index_map=(0,)),
        slice_sizes=(1,),
        mode=lax.GatherScatterMode.PROMISE_IN_BOUNDS)


@functools.partial(
    pl.kernel,
    out_type=jax.ShapeDtypeStruct((NT, E, B), jnp.float32),
    mesh=_sc_mesh,
    scratch_types=[
        pltpu.VMEM((E, B), jnp.float32),
        pltpu.VMEM((2, CHV, B), jnp.float32),
        pltpu.VMEM((2, ER, 128), jnp.float32),
        pltpu.SemaphoreType.DMA,
        pltpu.SemaphoreType.DMA,
        pltpu.SemaphoreType.DMA,
        pltpu.SemaphoreType.DMA,
    ],
)
def _sc_embed(emb_c_hbm, x_t_hbm, out_hbm, acc_ref, x_ref, e_ref,
              xsem0, xsem1, esem0, esem1):
    wid = lax.axis_index("s") * NC + lax.axis_index("c")
    base_ch = CH0 + wid * NCH_LO + jnp.minimum(wid, NCH_REM)
    n_ch = NCH_LO + jnp.where(wid < NCH_REM, 1, 0)

    def _start(c, buf, xsem, esem):
        j = base_ch + c
        pltpu.async_copy(x_t_hbm.at[pl.ds(j * CHV, CHV), :],
                         x_ref.at[buf], xsem)
        pltpu.async_copy(emb_c_hbm.at[j], e_ref.at[buf], esem)

    def _start_p(c, buf):
        @pl.when(buf == 0)
        def _():
            _start(c, 0, xsem0, esem0)

        @pl.when(buf == 1)
        def _():
            _start(c, 1, xsem1, esem1)

    def _wait_p(buf):
        @pl.when(buf == 0)
        def _():
            pltpu.make_async_copy(x_t_hbm.at[pl.ds(0, CHV), :],
                                  x_ref.at[0], xsem0).wait()
            pltpu.make_async_copy(emb_c_hbm.at[0], e_ref.at[0], esem0).wait()

        @pl.when(buf == 1)
        def _():
            pltpu.make_async_copy(x_t_hbm.at[pl.ds(0, CHV), :],
                                  x_ref.at[1], xsem1).wait()
            pltpu.make_async_copy(emb_c_hbm.at[0], e_ref.at[1], esem1).wait()

    zero = jnp.zeros((16,), jnp.float32)

    def _zero(g, carry):
        for e in range(E):
            acc_ref[e, pl.ds(g * 16, 16)] = zero
        return carry

    _start_p(0, 0)
    lax.fori_loop(0, B // 16, _zero, 0)

    def _chunk(c, carry):
        buf = lax.rem(c, 2)

        @pl.when(c + 1 < n_ch)
        def _():
            _start_p(c + 1, 1 - buf)

        _wait_p(buf)

        def _gblk(gb, carry2):
            b0 = gb * (G * 16)
            accs = tuple(acc_ref[e, pl.ds(b0 + g * 16, 16)]
                         for e in range(E) for g in range(G))

            for vb in range(CHV // 16):

                def _v(v, accs, vb=vb):
                    ilane = jnp.full((16,), v - vb * 16, jnp.int32)
                    xvs = [x_ref[buf, v, pl.ds(b0 + g * 16, 16)]
                           for g in range(G)]
                    new = []
                    i = 0
                    for e in range(E):
                        ev = e_ref[buf, (e * CHV + vb * 16) // 128,
                                   pl.ds((e * CHV + vb * 16) % 128, 16)]
                        s = _bcast(ev, ilane)
                        for g in range(G):
                            new.append(accs[i] + xvs[g] * s)
                            i += 1
                    return tuple(new)

                accs = lax.fori_loop(vb * 16, vb * 16 + 16, _v, accs)

            i = 0
            for e in range(E):
                for g in range(G):
                    acc_ref[e, pl.ds(b0 + g * 16, 16)] = accs[i]
                    i += 1
            return carry2

        lax.fori_loop(0, NGB, _gblk, 0)
        return carry

    lax.fori_loop(0, n_ch, _chunk, 0)

    pltpu.sync_copy(acc_ref, out_hbm.at[wid])


def _tc_body(e_ref, x_ref, o_ref):
    k = pl.program_id(0)

    @pl.when(k == 0)
    def _():
        o_ref[...] = jnp.zeros_like(o_ref)

    col = jax.lax.broadcasted_iota(jnp.int32, (E, VC), 1)
    em = jnp.where(k * VC + col < V_TC, e_ref[...], 0.0)
    o_ref[...] += jnp.dot(em, x_ref[...], preferred_element_type=jnp.float32)


def _tc_matmul(emb_t, x_t):
    return pl.pallas_call(
        _tc_body,
        grid=(NV_TC,),
        in_specs=[
            pl.BlockSpec((E, VC), lambda k: (0, k)),
            pl.BlockSpec((VC, B), lambda k: (k, 0)),
        ],
        out_specs=pl.BlockSpec((E, B), lambda k: (0, 0)),
        out_shape=jax.ShapeDtypeStruct((E, B), jnp.float32),
        compiler_params=pltpu.CompilerParams(
            dimension_semantics=("arbitrary",),
        ),
    )(emb_t, x_t)


@jax.jit
def kernel(x, embedding):
    emb_c = (embedding.T.reshape(E, NCHT, CHV).transpose(1, 0, 2)
             .reshape(NCHT, ER, 128))
    out_t = _sc_embed(emb_c, x.T).sum(axis=0)
    if V_TC:
        out_t = out_t + _tc_matmul(embedding.T, x.T)
    return out_t.T


# speedup vs baseline: 31.7803x; 1.2493x over previous
import functools

import jax
import jax.numpy as jnp
from jax import lax
from jax.experimental import pallas as pl
from jax.experimental.pallas import tpu as pltpu
from jax.experimental.pallas import tpu_sc as plsc

B, V, E = 1024, 100000, 16

CHV = 32
V_TC = 95904
NCHT = V // CHV
CH0 = V_TC // CHV
N_SC_CH = NCHT - CH0

VC = 2048
NV_TC = (V_TC + VC - 1) // VC

NC, NS = 2, 16
NT = NC * NS
NCH_LO = N_SC_CH // NT
NCH_REM = N_SC_CH - NCH_LO * NT
G = 2
NGB = (B // 16) // G
ER = E * CHV // 128

_sc_mesh = plsc.VectorSubcoreMesh(core_axis_name="c", subcore_axis_name="s")


def _bcast(vec, ilane):
    return lax.gather(
        vec, ilane[:, None],
        lax.GatherDimensionNumbers(offset_dims=(), collapsed_slice_dims=(0,),
                                   start_index_map=(0,)),
        slice_sizes=(1,),
        mode=lax.GatherScatterMode.PROMISE_IN_BOUNDS)


@functools.partial(
    pl.kernel,
    out_type=jax.ShapeDtypeStruct((NT, E, B), jnp.float32),
    mesh=_sc_mesh,
    scratch_types=[
        pltpu.VMEM((E, B), jnp.float32),
        pltpu.VMEM((2, CHV, B), jnp.float32),
        pltpu.VMEM((2, ER, 128), jnp.float32),
        pltpu.SemaphoreType.DMA,
        pltpu.SemaphoreType.DMA,
        pltpu.SemaphoreType.DMA,
        pltpu.SemaphoreType.DMA,
    ],
)
def _sc_embed(emb_c_hbm, x_t_hbm, out_hbm, acc_ref, x_ref, e_ref,
              xsem0, xsem1, esem0, esem1):
    wid = lax.axis_index("s") * NC + lax.axis_index("c")
    base_ch = CH0 + wid * NCH_LO + jnp.minimum(wid, NCH_REM)
    n_ch = NCH_LO + jnp.where(wid < NCH_REM, 1, 0)

    def _start(c, buf, xsem, esem):
        j = base_ch + c
        pltpu.async_copy(x_t_hbm.at[pl.ds(j * CHV, CHV), :],
                         x_ref.at[buf], xsem)
        pltpu.async_copy(emb_c_hbm.at[j - CH0], e_ref.at[buf], esem)

    def _start_p(c, buf):
        @pl.when(buf == 0)
        def _():
            _start(c, 0, xsem0, esem0)

        @pl.when(buf == 1)
        def _():
            _start(c, 1, xsem1, esem1)

    def _wait_p(buf):
        @pl.when(buf == 0)
        def _():
            pltpu.make_async_copy(x_t_hbm.at[pl.ds(0, CHV), :],
                                  x_ref.at[0], xsem0).wait()
            pltpu.make_async_copy(emb_c_hbm.at[0], e_ref.at[0], esem0).wait()

        @pl.when(buf == 1)
        def _():
            pltpu.make_async_copy(x_t_hbm.at[pl.ds(0, CHV), :],
                                  x_ref.at[1], xsem1).wait()
            pltpu.make_async_copy(emb_c_hbm.at[0], e_ref.at[1], esem1).wait()

    zero = jnp.zeros((16,), jnp.float32)

    def _zero(g, carry):
        for e in range(E):
            acc_ref[e, pl.ds(g * 16, 16)] = zero
        return carry

    _start_p(0, 0)
    lax.fori_loop(0, B // 16, _zero, 0)

    def _chunk(c, carry):
        buf = lax.rem(c, 2)

        @pl.when(c + 1 < n_ch)
        def _():
            _start_p(c + 1, 1 - buf)

        _wait_p(buf)

        def _gblk(gb, carry2):
            b0 = gb * (G * 16)
            accs = tuple(acc_ref[e, pl.ds(b0 + g * 16, 16)]
                         for e in range(E) for g in range(G))

            for vb in range(CHV // 16):

                def _v(v, accs, vb=vb):
                    ilane = jnp.full((16,), v - vb * 16, jnp.int32)
                    xvs = [x_ref[buf, v, pl.ds(b0 + g * 16, 16)]
                           for g in range(G)]
                    new = []
                    i = 0
                    for e in range(E):
                        ev = e_ref[buf, (e * CHV + vb * 16) // 128,
                                   pl.ds((e * CHV + vb * 16) % 128, 16)]
                        s = _bcast(ev, ilane)
                        for g in range(G):
                            new.append(accs[i] + xvs[g] * s)
                            i += 1
                    return tuple(new)

                accs = lax.fori_loop(vb * 16, vb * 16 + 16, _v, accs)

            i = 0
            for e in range(E):
                for g in range(G):
                    acc_ref[e, pl.ds(b0 + g * 16, 16)] = accs[i]
                    i += 1
            return carry2

        lax.fori_loop(0, NGB, _gblk, 0)
        return carry

    lax.fori_loop(0, n_ch, _chunk, 0)

    pltpu.sync_copy(acc_ref, out_hbm.at[wid])


def _tc_body(e_ref, x_ref, o_ref):
    k = pl.program_id(0)

    @pl.when(k == 0)
    def _():
        o_ref[...] = jnp.zeros_like(o_ref)

    col = jax.lax.broadcasted_iota(jnp.int32, (E, VC), 1)
    em = jnp.where(k * VC + col < V_TC, e_ref[...], 0.0)
    o_ref[...] += jnp.dot(em, x_ref[...], preferred_element_type=jnp.float32)


def _tc_matmul(emb_t, x_t):
    return pl.pallas_call(
        _tc_body,
        grid=(NV_TC,),
        in_specs=[
            pl.BlockSpec((E, VC), lambda k: (0, k)),
            pl.BlockSpec((VC, B), lambda k: (k, 0)),
        ],
        out_specs=pl.BlockSpec((E, B), lambda k: (0, 0)),
        out_shape=jax.ShapeDtypeStruct((E, B), jnp.float32),
        compiler_params=pltpu.CompilerParams(
            dimension_semantics=("arbitrary",),
        ),
    )(emb_t, x_t)


@jax.jit
def kernel(x, embedding):
    emb_c = (embedding.T[:, V_TC:].reshape(E, N_SC_CH, CHV).transpose(1, 0, 2)
             .reshape(N_SC_CH, ER, 128))
    out_t = _sc_embed(emb_c, x.T).sum(axis=0)
    if V_TC:
        out_t = out_t + _tc_matmul(embedding.T, x.T)
    return out_t.T


# speedup vs baseline: 37.0455x; 1.1657x over previous
import jax
import jax.numpy as jnp
from jax.experimental import pallas as pl
from jax.experimental.pallas import tpu as pltpu

B, V, E = 1024, 100000, 16

VC = 2048
NV = (V + VC - 1) // VC


def _mm_body(e_ref, x_ref, o_ref):
    k = pl.program_id(0)

    @pl.when(k == 0)
    def _():
        o_ref[...] = jnp.zeros_like(o_ref)

    col = jax.lax.broadcasted_iota(jnp.int32, (E, VC), 1)
    em = jnp.where(k * VC + col < V, e_ref[...], 0.0)
    o_ref[...] += jnp.dot(em, x_ref[...], preferred_element_type=jnp.float32)


@jax.jit
def kernel(x, embedding):
    out_t = pl.pallas_call(
        _mm_body,
        grid=(NV,),
        in_specs=[
            pl.BlockSpec((E, VC), lambda k: (0, k)),
            pl.BlockSpec((VC, B), lambda k: (k, 0)),
        ],
        out_specs=pl.BlockSpec((E, B), lambda k: (0, 0)),
        out_shape=jax.ShapeDtypeStruct((E, B), jnp.float32),
        compiler_params=pltpu.CompilerParams(
            dimension_semantics=("arbitrary",),
        ),
    )(embedding.T, x.T)
    return out_t.T


# speedup vs baseline: 37.3859x; 1.0092x over previous
import jax
import jax.numpy as jnp
from jax.experimental import pallas as pl
from jax.experimental.pallas import tpu as pltpu

B, V, E = 1024, 100000, 16

VC = 3072
NV = (V + VC - 1) // VC


def _mm_body(e_ref, x_ref, o_ref):
    k = pl.program_id(0)

    @pl.when(k == 0)
    def _():
        o_ref[...] = jnp.zeros_like(o_ref)

    col = jax.lax.broadcasted_iota(jnp.int32, (E, VC), 1)
    em = jnp.where(k * VC + col < V, e_ref[...], 0.0)
    o_ref[...] += jnp.dot(em, x_ref[...], preferred_element_type=jnp.float32)


@jax.jit
def kernel(x, embedding):
    out_t = pl.pallas_call(
        _mm_body,
        grid=(NV,),
        in_specs=[
            pl.BlockSpec((E, VC), lambda k: (0, k)),
            pl.BlockSpec((VC, B), lambda k: (k, 0)),
        ],
        out_specs=pl.BlockSpec((E, B), lambda k: (0, 0)),
        out_shape=jax.ShapeDtypeStruct((E, B), jnp.float32),
        compiler_params=pltpu.CompilerParams(
            dimension_semantics=("arbitrary",),
        ),
    )(embedding.T, x.T)
    return out_t.T


# speedup vs baseline: 37.4473x; 1.0016x over previous
import jax
import jax.numpy as jnp
from jax.experimental import pallas as pl
from jax.experimental.pallas import tpu as pltpu

B, V, E = 1024, 100000, 16

VC = 2048
NV = (V + VC - 1) // VC


def _mm_body(e_ref, x_ref, o_ref):
    k = pl.program_id(0)

    @pl.when(k == 0)
    def _():
        o_ref[...] = jnp.zeros_like(o_ref)

    col = jax.lax.broadcasted_iota(jnp.int32, (E, VC), 1)
    em = jnp.where(k * VC + col < V, e_ref[...], 0.0)
    o_ref[...] += jnp.dot(em, x_ref[...], preferred_element_type=jnp.float32)


@jax.jit
def kernel(x, embedding):
    out_t = pl.pallas_call(
        _mm_body,
        grid=(NV,),
        in_specs=[
            pl.BlockSpec((E, VC), lambda k: (0, k)),
            pl.BlockSpec((VC, B), lambda k: (k, 0)),
        ],
        out_specs=pl.BlockSpec((E, B), lambda k: (0, 0)),
        out_shape=jax.ShapeDtypeStruct((E, B), jnp.float32),
        compiler_params=pltpu.CompilerParams(
            dimension_semantics=("arbitrary",),
        ),
    )(embedding.T, x.T)
    return out_t.T
